# Initial kernel scaffold; baseline (speedup 1.0000x reference)
import jax
import jax.numpy as jnp
from jax.experimental import pallas as pl


def kernel(tokens, table):
    raise NotImplementedError("write your pallas kernel here")


# speedup vs baseline: 4.6950x; 4.6950x over previous
import functools
import math

import jax
import jax.numpy as jnp
from jax import lax
from jax.experimental import pallas as pl
from jax.experimental.pallas import tpu as pltpu
from jax.experimental.pallas import tpu_sc as plsc

EMBED_DIM = 32
SCALE = math.sqrt(EMBED_DIM)

NUM_CORES = 2
NUM_SUBCORES = 16
NW = NUM_CORES * NUM_SUBCORES

SEQ = 16384 * 200
BPW = SEQ // NW
G = 128
K = 16
CHUNK = K * G
NCHUNKS = BPW // CHUNK


def _make_sc_embed():
    mesh = plsc.VectorSubcoreMesh(core_axis_name="c", subcore_axis_name="s")

    @functools.partial(
        pl.kernel,
        mesh=mesh,
        out_type=jax.ShapeDtypeStruct((SEQ, EMBED_DIM), jnp.float32),
        scratch_types=[
            pltpu.VMEM((K, G), jnp.int32),
            pltpu.VMEM((CHUNK, EMBED_DIM), jnp.float32),
            pltpu.SemaphoreType.DMA,
        ],
        compiler_params=pltpu.CompilerParams(use_tc_tiling_on_sc=False),
    )
    def embed(idx_hbm, table_hbm, out_hbm, idx_v, rows_v, sem):
        wid = lax.axis_index("s") * NUM_CORES + lax.axis_index("c")
        row0 = wid * (BPW // G)

        def chunk_body(g, _):
            pltpu.sync_copy(idx_hbm.at[pl.ds(row0 + g * K, K)], idx_v)
            copies = []
            for j in range(K):
                copies.append(
                    pltpu.async_copy(
                        table_hbm.at[idx_v.at[j]],
                        rows_v.at[pl.ds(j * G, G)],
                        sem,
                    )
                )
            for c in copies:
                c.wait()

            def scale_body(i, _):
                rows_v[i, 0:16] = rows_v[i, 0:16] * SCALE
                rows_v[i, 16:32] = rows_v[i, 16:32] * SCALE
                return _

            lax.fori_loop(0, CHUNK, scale_body, None, unroll=4)

            pltpu.sync_copy(
                rows_v, out_hbm.at[pl.ds(wid * BPW + g * CHUNK, CHUNK)]
            )
            return _

        lax.fori_loop(0, NCHUNKS, chunk_body, None)

    return embed


_sc_embed = _make_sc_embed()


@jax.jit
def _embed(tokens, table):
    idx = tokens.reshape(SEQ // G, G).astype(jnp.int32)
    out = _sc_embed(idx, table)
    return out.reshape(tokens.shape + (EMBED_DIM,))


def kernel(tokens, table):
    return _embed(tokens, table)


# speedup vs baseline: 5.0099x; 1.0671x over previous
import functools
import math

import jax
import jax.numpy as jnp
from jax import lax
from jax.experimental import pallas as pl
from jax.experimental.pallas import tpu as pltpu
from jax.experimental.pallas import tpu_sc as plsc

EMBED_DIM = 32
SCALE = math.sqrt(EMBED_DIM)

NUM_CORES = 2
NUM_SUBCORES = 16
NW = NUM_CORES * NUM_SUBCORES

SEQ = 16384 * 200
BPW = SEQ // NW
G = 128
K = 10
CHUNK = K * G
NCHUNKS = BPW // CHUNK


def _make_sc_embed():
    mesh = plsc.VectorSubcoreMesh(core_axis_name="c", subcore_axis_name="s")

    @functools.partial(
        pl.kernel,
        mesh=mesh,
        out_type=jax.ShapeDtypeStruct((SEQ, EMBED_DIM), jnp.float32),
        scratch_types=[
            pltpu.VMEM((2, K, G), jnp.int32),
            pltpu.VMEM((2, CHUNK, EMBED_DIM), jnp.float32),
            pltpu.SemaphoreType.DMA,
            pltpu.SemaphoreType.DMA,
            pltpu.SemaphoreType.DMA,
        ],
        compiler_params=pltpu.CompilerParams(use_tc_tiling_on_sc=False),
    )
    def embed(idx_hbm, table_hbm, out_hbm, idx_v, rows_v, s_idx, s_g, s_out):
        wid = lax.axis_index("s") * NUM_CORES + lax.axis_index("c")
        row0 = wid * (BPW // G)
        out0 = wid * BPW

        def idx_fetch(g, b):
            return pltpu.async_copy(
                idx_hbm.at[pl.ds(row0 + g * K, K)], idx_v.at[b], s_idx
            )

        def fire_gathers(b):
            for j in range(K):
                pltpu.async_copy(
                    table_hbm.at[idx_v.at[b, j]],
                    rows_v.at[b, pl.ds(j * G, G)],
                    s_g,
                )

        def drain_gathers(b):
            for j in range(K):
                pltpu.make_async_copy(
                    table_hbm.at[idx_v.at[b, j]],
                    rows_v.at[b, pl.ds(j * G, G)],
                    s_g,
                ).wait()

        def wait_out(b):
            pltpu.make_async_copy(
                rows_v.at[b], out_hbm.at[pl.ds(0, CHUNK)], s_out
            ).wait()

        def scale(b):
            def body(i, carry):
                rows_v[b, i, 0:16] = rows_v[b, i, 0:16] * SCALE
                rows_v[b, i, 16:32] = rows_v[b, i, 16:32] * SCALE
                return carry

            lax.fori_loop(0, CHUNK, body, None, unroll=8)

        idx_fetch(0, 0).wait()
        fire_gathers(0)
        idx_fetch(1, 1)

        def body(i, carry):
            for b in (0, 1):
                g = 2 * i + b
                drain_gathers(b)

                @pl.when(g + 2 < NCHUNKS)
                def _():
                    idx_fetch(g + 2, b)

                @pl.when(g >= 1)
                def _():
                    wait_out(1 - b)

                @pl.when(g + 1 < NCHUNKS)
                def _():
                    pltpu.make_async_copy(
                        idx_hbm.at[pl.ds(0, K)], idx_v.at[1 - b], s_idx
                    ).wait()
                    fire_gathers(1 - b)

                scale(b)
                pltpu.async_copy(
                    rows_v.at[b],
                    out_hbm.at[pl.ds(out0 + g * CHUNK, CHUNK)],
                    s_out,
                )
            return carry

        lax.fori_loop(0, NCHUNKS // 2, body, None)
        wait_out(1)

    return embed


_sc_embed = _make_sc_embed()


@jax.jit
def _embed(tokens, table):
    idx = tokens.reshape(SEQ // G, G).astype(jnp.int32)
    out = _sc_embed(idx, table)
    return out.reshape(tokens.shape + (EMBED_DIM,))


def kernel(tokens, table):
    return _embed(tokens, table)
